# baseline (speedup 1.0000x reference)
import functools

import jax
import jax.numpy as jnp
from jax.experimental import pallas as pl
from jax.experimental.pallas import tpu as pltpu

_N = 36864
_K = 1024
_D = 64
_BLOCK = 4096


def _body(x_ref, w_ref, zq_ref, probs_ref):
    x = x_ref[...]
    w = w_ref[...]
    xsq = jnp.sum(x * x, axis=1, keepdims=True)
    wsq = jnp.sum(w * w, axis=1)
    xw = jax.lax.dot_general(
        x, w, (((1,), (1,)), ((), ())), preferred_element_type=jnp.float32
    )
    dist = xsq + wsq - 2.0 * xw
    ncols = _K // 128
    cols = [dist[:, j * 128:(j + 1) * 128] for j in range(ncols)]
    vt = list(cols)
    jt = [jnp.full(cols[0].shape, j, jnp.int32) for j in range(ncols)]
    while len(vt) > 1:
        nv, nj = [], []
        for p in range(0, len(vt), 2):
            take_b = vt[p + 1] < vt[p]
            nv.append(jnp.minimum(vt[p], vt[p + 1]))
            nj.append(jnp.where(take_b, jt[p + 1], jt[p]))
        vt, jt = nv, nj
    lane = jax.lax.broadcasted_iota(jnp.int32, cols[0].shape, 1)
    gidx = jt[0] * 128 + lane
    minval = jnp.min(vt[0], axis=1, keepdims=True)
    idx = jnp.min(jnp.where(vt[0] == minval, gidx, _K), axis=1, keepdims=True)
    iota = jax.lax.broadcasted_iota(jnp.int32, dist.shape, 1)
    onehot = (iota == idx).astype(jnp.float32)
    probs_ref[...] = onehot
    dn = (((1,), (0,)), ((), ()))
    zq_ref[...] = jax.lax.dot_general(
        onehot, w, dn, preferred_element_type=jnp.float32
    )


@functools.partial(jax.jit, static_argnames=())
def kernel(x, W):
    n, d = x.shape
    k = W.shape[0]
    grid = (n // _BLOCK,)
    zq, probs = pl.pallas_call(
        _body,
        grid=grid,
        in_specs=[
            pl.BlockSpec((_BLOCK, d), lambda i: (i, 0)),
            pl.BlockSpec((k, d), lambda i: (0, 0)),
        ],
        out_specs=[
            pl.BlockSpec((_BLOCK, d), lambda i: (i, 0)),
            pl.BlockSpec((_BLOCK, k), lambda i: (i, 0)),
        ],
        out_shape=[
            jax.ShapeDtypeStruct((n, d), jnp.float32),
            jax.ShapeDtypeStruct((n, k), jnp.float32),
        ],
        compiler_params=pltpu.CompilerParams(
            vmem_limit_bytes=120 * 1024 * 1024,
        ),
    )(x, W)
    return (zq, probs)


# speedup vs baseline: 1.0079x; 1.0079x over previous
import functools

import jax
import jax.numpy as jnp
from jax.experimental import pallas as pl
from jax.experimental.pallas import tpu as pltpu

_N = 36864
_K = 1024
_D = 64
_BLOCK = 3072


def _body(x_ref, w_ref, zq_ref, probs_ref):
    x = x_ref[...]
    w = w_ref[...]
    xsq = jnp.sum(x * x, axis=1, keepdims=True)
    wsq = jnp.sum(w * w, axis=1)
    xw = jax.lax.dot_general(
        x, w, (((1,), (1,)), ((), ())), preferred_element_type=jnp.float32
    )
    dist = xsq + wsq - 2.0 * xw
    ncols = _K // 128
    cols = [dist[:, j * 128:(j + 1) * 128] for j in range(ncols)]
    vt = list(cols)
    jt = [jnp.full(cols[0].shape, j, jnp.int32) for j in range(ncols)]
    while len(vt) > 1:
        nv, nj = [], []
        for p in range(0, len(vt), 2):
            take_b = vt[p + 1] < vt[p]
            nv.append(jnp.minimum(vt[p], vt[p + 1]))
            nj.append(jnp.where(take_b, jt[p + 1], jt[p]))
        vt, jt = nv, nj
    lane = jax.lax.broadcasted_iota(jnp.int32, cols[0].shape, 1)
    gidx = jt[0] * 128 + lane
    minval = jnp.min(vt[0], axis=1, keepdims=True)
    idx = jnp.min(jnp.where(vt[0] == minval, gidx, _K), axis=1, keepdims=True)
    iota = jax.lax.broadcasted_iota(jnp.int32, dist.shape, 1)
    onehot = (iota == idx).astype(jnp.float32)
    probs_ref[...] = onehot
    dn = (((1,), (0,)), ((), ()))
    zq_ref[...] = jax.lax.dot_general(
        onehot, w, dn, preferred_element_type=jnp.float32
    )


@functools.partial(jax.jit, static_argnames=())
def kernel(x, W):
    n, d = x.shape
    k = W.shape[0]
    grid = (n // _BLOCK,)
    zq, probs = pl.pallas_call(
        _body,
        grid=grid,
        in_specs=[
            pl.BlockSpec((_BLOCK, d), lambda i: (i, 0)),
            pl.BlockSpec((k, d), lambda i: (0, 0)),
        ],
        out_specs=[
            pl.BlockSpec((_BLOCK, d), lambda i: (i, 0)),
            pl.BlockSpec((_BLOCK, k), lambda i: (i, 0)),
        ],
        out_shape=[
            jax.ShapeDtypeStruct((n, d), jnp.float32),
            jax.ShapeDtypeStruct((n, k), jnp.float32),
        ],
        compiler_params=pltpu.CompilerParams(
            vmem_limit_bytes=120 * 1024 * 1024,
        ),
    )(x, W)
    return (zq, probs)
